# baseline (speedup 1.0000x reference)
import functools

import jax
import jax.numpy as jnp
from jax import lax
from jax.experimental import pallas as pl
from jax.experimental.pallas import tpu as pltpu
from jax.experimental.pallas import tpu_sc as plsc

_NCORES = 2
_NSUB = 16
_NW = _NCORES * _NSUB
_LANES = 16

_CHUNK = 64
_CPW = 5
_NBUCKET = _NW * _CPW
_NPAD = _NBUCKET * _CHUNK
_CAP = 2048
_SB = 1600
_GB = 64


def _splat_lane(v, j):
    idx = jnp.full((_LANES, 1), j, jnp.int32)
    dnums = lax.GatherDimensionNumbers(
        offset_dims=(), collapsed_slice_dims=(0,), start_index_map=(0,))
    return lax.gather(v, idx, dnums, (1,),
                      mode=lax.GatherScatterMode.PROMISE_IN_BOUNDS)


def _tile8(b):
    return jnp.tile(b.reshape(1, -1), (8, 1))


def _dotT(a, b):
    return lax.dot_general(a, b, (((1,), (1,)), ((), ())),
                           preferred_element_type=jnp.float32)


def _tc_matmul(xin, w, b8, relu, l2n):
    m, k = xin.shape
    dout = w.shape[0]
    bm = 2000
    grid = m // bm

    def body(x_ref, w_ref, b_ref, o_ref):
        t = _dotT(x_ref[...], w_ref[...]) + b_ref[0:1, :]
        if relu:
            t = jnp.maximum(t, 0.0)
        if l2n:
            nrm = jnp.sqrt(jnp.sum(t * t, axis=1, keepdims=True))
            t = t / jnp.maximum(nrm, 1e-12)
        o_ref[...] = t

    return pl.pallas_call(
        body,
        grid=(grid,),
        in_specs=[
            pl.BlockSpec((bm, k), lambda i: (i, 0)),
            pl.BlockSpec((dout, k), lambda i: (0, 0)),
            pl.BlockSpec((8, dout), lambda i: (0, 0)),
        ],
        out_specs=pl.BlockSpec((bm, dout), lambda i: (i, 0)),
        out_shape=jax.ShapeDtypeStruct((m, dout), jnp.float32),
    )(xin, w, b8)


def _tc_update(h, agg, ms, mn, extra):
    m, dh = h.shape
    bm = 2000
    grid = m // bm

    def body(h_ref, a_ref, ms_ref, mn_ref, e_ref, o_ref):
        t = _dotT(h_ref[...], ms_ref[...]) + _dotT(a_ref[...], mn_ref[...])
        t = t + e_ref[...]
        t = jnp.maximum(t, 0.0)
        nrm = jnp.sqrt(jnp.sum(t * t, axis=1, keepdims=True))
        o_ref[...] = t / jnp.maximum(nrm, 1e-12)

    return pl.pallas_call(
        body,
        grid=(grid,),
        in_specs=[
            pl.BlockSpec((bm, dh), lambda i: (i, 0)),
            pl.BlockSpec((bm, dh), lambda i: (i, 0)),
            pl.BlockSpec((dh, dh), lambda i: (0, 0)),
            pl.BlockSpec((dh, dh), lambda i: (0, 0)),
            pl.BlockSpec((bm, dh), lambda i: (i, 0)),
        ],
        out_specs=pl.BlockSpec((bm, dh), lambda i: (i, 0)),
        out_shape=jax.ShapeDtypeStruct((m, dh), jnp.float32),
    )(h, agg, ms, mn, extra)


def _tc_prep(ws, wn, wu, bs8, bn8, bu8):
    dh = ws.shape[0]

    def body(ws_ref, wn_ref, wu_ref, bs_ref, bn_ref, bu_ref,
             ms_ref, mn_ref, c_ref, d_ref):
        wua = wu_ref[:, :dh]
        wub = wu_ref[:, dh:]
        ms_ref[...] = lax.dot_general(wua, ws_ref[...],
                                      (((1,), (0,)), ((), ())),
                                      preferred_element_type=jnp.float32)
        mn_ref[...] = lax.dot_general(wub, wn_ref[...],
                                      (((1,), (0,)), ((), ())),
                                      preferred_element_type=jnp.float32)
        c_ref[...] = _dotT(bs_ref[...], wua) + bu_ref[...]
        d_ref[...] = _dotT(bn_ref[...], wub)

    return pl.pallas_call(
        body,
        out_shape=(
            jax.ShapeDtypeStruct((dh, dh), jnp.float32),
            jax.ShapeDtypeStruct((dh, dh), jnp.float32),
            jax.ShapeDtypeStruct((8, dh), jnp.float32),
            jax.ShapeDtypeStruct((8, dh), jnp.float32),
        ),
    )(ws, wn, wu, bs8, bn8, bu8)


def _sc_mesh():
    return plsc.VectorSubcoreMesh(core_axis_name="c", subcore_axis_name="s",
                                  num_cores=_NCORES, num_subcores=_NSUB)


def _make_bucket_kernel(e_total):
    t_outer = e_total // _SB
    g_inner = _SB // _LANES

    @functools.partial(
        pl.kernel,
        mesh=_sc_mesh(),
        compiler_params=pltpu.CompilerParams(needs_layout_passes=False),
        out_type=(
            jax.ShapeDtypeStruct((_NBUCKET * _CAP,), jnp.int32),
            jax.ShapeDtypeStruct((_NBUCKET * _CAP,), jnp.int32),
            jax.ShapeDtypeStruct((_NBUCKET * _LANES,), jnp.int32),
        ),
        scratch_types=[
            pltpu.VMEM((_SB,), jnp.int32),
            pltpu.VMEM((_SB,), jnp.int32),
            pltpu.VMEM((_CPW * _CAP + _LANES,), jnp.int32),
            pltpu.VMEM((_CPW * _CAP + _LANES,), jnp.int32),
            pltpu.VMEM((_CPW * _LANES,), jnp.int32),
        ],
    )
    def bucket(src_hbm, dst_hbm, bsrc_hbm, bdst_hbm, bcnt_hbm,
               sbuf, dbuf, bbs, bbd, cbuf):
        wid = lax.axis_index("s") * _NCORES + lax.axis_index("c")
        kc_base = wid * _CPW
        lane = lax.broadcasted_iota(jnp.int32, (_LANES,), 0)
        trash = _CPW * _CAP + lane

        def group(g, cnts):
            o = g * _LANES
            dstv = dbuf[pl.ds(o, _LANES)]
            srcv = sbuf[pl.ds(o, _LANES)]
            chunkv = jnp.right_shift(dstv, 6)
            dstloc = jnp.bitwise_and(dstv, _CHUNK - 1)
            out = []
            for k in range(_CPW):
                ck = cnts[k]
                m = chunkv == (kc_base + k)
                pref = plsc.cumsum(m.astype(jnp.int32))
                off = k * _CAP + jnp.minimum(ck, _CAP - _LANES)
                pos = jnp.where(m, off + pref - 1, trash)
                plsc.store_scatter(bbs, [pos], srcv)
                plsc.store_scatter(bbd, [pos], dstloc)
                pc = plsc.all_reduce_population_count(m)
                out.append(ck + pc)
            return tuple(out)

        def outer(t, cnts):
            pltpu.sync_copy(src_hbm.at[pl.ds(t * _SB, _SB)], sbuf)
            pltpu.sync_copy(dst_hbm.at[pl.ds(t * _SB, _SB)], dbuf)
            return lax.fori_loop(0, g_inner, group, cnts)

        zero16 = jnp.zeros((_LANES,), jnp.int32)
        cnts = lax.fori_loop(0, t_outer, outer, (zero16,) * _CPW)

        zdst = jnp.full((_LANES,), _DUMMY_DST, jnp.int32)
        for k in range(_CPW):
            ck = jnp.minimum(cnts[k], _CAP - _LANES)
            for t in range(8):
                pos = k * _CAP + jnp.minimum(ck + _LANES * t,
                                             _CAP - _LANES) + lane
                plsc.store_scatter(bbs, [pos], zero16)
                plsc.store_scatter(bbd, [pos], zdst)
            cbuf[pl.ds(k * _LANES, _LANES)] = ck

        pltpu.sync_copy(bbs.at[pl.ds(0, _CPW * _CAP)],
                        bsrc_hbm.at[pl.ds(wid * _CPW * _CAP, _CPW * _CAP)])
        pltpu.sync_copy(bbd.at[pl.ds(0, _CPW * _CAP)],
                        bdst_hbm.at[pl.ds(wid * _CPW * _CAP, _CPW * _CAP)])
        pltpu.sync_copy(cbuf, bcnt_hbm.at[pl.ds(wid * _CPW * _LANES,
                                                _CPW * _LANES)])

    return bucket


_DUMMY_DST = 8192
_DEG_LEN = _CPW * _CHUNK + _LANES


def _make_agg_kernel(n_nodes, dh, with_deg):
    gb = 16
    nring = 4
    acc_len = (_CHUNK + 1) * dh

    out_type = [jax.ShapeDtypeStruct((_NPAD * dh,), jnp.float32)]
    if with_deg:
        out_type.append(jax.ShapeDtypeStruct((_NPAD,), jnp.float32))

    @functools.partial(
        pl.kernel,
        mesh=_sc_mesh(),
        compiler_params=pltpu.CompilerParams(needs_layout_passes=False),
        out_type=tuple(out_type),
        scratch_types=[
            pltpu.VMEM((_CAP,), jnp.int32),
            pltpu.VMEM((_CAP,), jnp.int32),
            [pltpu.VMEM((gb, dh), jnp.float32) for _ in range(nring)],
            pltpu.VMEM((acc_len,), jnp.float32),
            pltpu.VMEM((_LANES,), jnp.int32),
            pltpu.VMEM((_DEG_LEN,), jnp.float32),
            [pltpu.SemaphoreType.DMA for _ in range(nring)],
        ],
    )
    def agg(h_hbm, bsrc_hbm, bdst_hbm, bcnt_hbm, zeros_hbm,
            *out_and_scratch):
        if with_deg:
            agg_hbm, deg_hbm = out_and_scratch[0], out_and_scratch[1]
            rest = out_and_scratch[2:]
        else:
            agg_hbm = out_and_scratch[0]
            rest = out_and_scratch[1:]
        ibuf, dbuf, rings, acc, cntbuf, degacc, sems = rest

        cid = lax.axis_index("c")
        sid = lax.axis_index("s")
        wid = sid * _NCORES + cid
        lane = lax.broadcasted_iota(jnp.int32, (_LANES,), 0)
        zeros16 = jnp.zeros((_LANES,), jnp.float32)
        ones16 = jnp.ones((_LANES,), jnp.float32)

        def start(b, rows, sem):
            off = jnp.minimum(b * gb, _CAP - gb)
            return pltpu.async_copy(h_hbm.at[ibuf.at[pl.ds(off, gb)]],
                                    rows, sem)

        def wait(rows, sem):
            pltpu.make_async_copy(h_hbm.at[ibuf.at[pl.ds(0, gb)]],
                                  rows, sem).wait()

        cconst = [lane + _LANES * c for c in range(dh // _LANES)]

        def accumulate(bbase, rows, k):
            dv = dbuf[pl.ds(bbase, _LANES)]
            dvm = jnp.minimum(dv, _CHUNK) * dh
            if with_deg:
                di = jnp.minimum(k * _CHUNK + dv, _CPW * _CHUNK)
                plsc.addupdate_scatter(degacc, [di], ones16)
            for j in range(_LANES):
                base = _splat_lane(dvm, j)
                for c in range(dh // _LANES):
                    plsc.addupdate_scatter(
                        acc, [base + cconst[c]],
                        rows[j, pl.ds(c * _LANES, _LANES)])

        if with_deg:
            for t in range(_DEG_LEN // _LANES):
                degacc[pl.ds(t * _LANES, _LANES)] = zeros16

        def do_bucket(k, _):
            bucket = wid * _CPW + k
            pltpu.sync_copy(bcnt_hbm.at[pl.ds(bucket * _LANES, _LANES)],
                            cntbuf)
            cnt = jnp.max(cntbuf[...])
            pltpu.sync_copy(bsrc_hbm.at[pl.ds(bucket * _CAP, _CAP)], ibuf)
            pltpu.sync_copy(bdst_hbm.at[pl.ds(bucket * _CAP, _CAP)], dbuf)
            pltpu.sync_copy(zeros_hbm, acc)
            nquads = (cnt + nring * gb - 1) // (nring * gb)
            for b in range(nring - 1):
                start(b, rings[b], sems[b])

            def quad(tt, _):
                for b in range(nring):
                    bi = nring * tt + b
                    start(bi + nring - 1, rings[(b + nring - 1) % nring],
                          sems[(b + nring - 1) % nring])
                    wait(rings[b], sems[b])
                    accumulate(bi * gb, rings[b], k)
                return 0

            lax.fori_loop(0, nquads, quad, 0)
            for b in range(nring - 1):
                wait(rings[b], sems[b])
            pltpu.sync_copy(acc.at[pl.ds(0, _CHUNK * dh)],
                            agg_hbm.at[pl.ds(bucket * _CHUNK * dh,
                                             _CHUNK * dh)])
            return 0

        lax.fori_loop(0, _CPW, do_bucket, 0)
        if with_deg:
            pltpu.sync_copy(degacc.at[pl.ds(0, _CPW * _CHUNK)],
                            deg_hbm.at[pl.ds(wid * _CPW * _CHUNK,
                                             _CPW * _CHUNK)])

    return agg


def kernel(x, edge_index, W_in, b_in, Ws0, bs0, Wn0, bn0, Wu0, bu0,
           Ws1, bs1, Wn1, bn1, Wu1, bu1, W_out, b_out):
    n, din = x.shape
    dh = W_in.shape[0]
    e_total = edge_index.shape[1]

    src = edge_index[0]
    dst = edge_index[1]

    ms0, mn0, c0, d0 = _tc_prep(Ws0, Wn0, Wu0, _tile8(bs0), _tile8(bn0),
                                _tile8(bu0))
    ms1, mn1, c1, d1 = _tc_prep(Ws1, Wn1, Wu1, _tile8(bs1), _tile8(bn1),
                                _tile8(bu1))

    h = _tc_matmul(x, W_in, _tile8(b_in), relu=True, l2n=False)

    bsrc, bdst, bcnt = _make_bucket_kernel(e_total)(src, dst)

    zeros_acc = jnp.zeros(((_CHUNK + 1) * dh,), jnp.float32)

    agg0_flat, deg_pad = _make_agg_kernel(n, dh, True)(
        h, bsrc, bdst, bcnt, zeros_acc)
    agg0 = agg0_flat.reshape(_NPAD, dh)[:n]
    deg = deg_pad[:n]

    extra0 = deg[:, None] * d0[0] + c0[0]
    h = _tc_update(h, agg0, ms0, mn0, extra0)

    (agg1_flat,) = _make_agg_kernel(n, dh, False)(
        h, bsrc, bdst, bcnt, zeros_acc)
    agg1 = agg1_flat.reshape(_NPAD, dh)[:n]

    extra1 = deg[:, None] * d1[0] + c1[0]
    h = _tc_update(h, agg1, ms1, mn1, extra1)

    emb = _tc_matmul(h, W_out, _tile8(b_out), relu=False, l2n=True)
    return emb


# speedup vs baseline: 1.2219x; 1.2219x over previous
import functools

import jax
import jax.numpy as jnp
from jax import lax
from jax.experimental import pallas as pl
from jax.experimental.pallas import tpu as pltpu
from jax.experimental.pallas import tpu_sc as plsc

_NCORES = 2
_NSUB = 16
_NW = _NCORES * _NSUB
_LANES = 16

_CHUNK = 64
_CPW = 5
_NBUCKET = _NW * _CPW
_NPAD = _NBUCKET * _CHUNK
_CAP = 2048
_SB = 1600
_GB = 64


def _splat_lane(v, j):
    idx = jnp.full((_LANES, 1), j, jnp.int32)
    dnums = lax.GatherDimensionNumbers(
        offset_dims=(), collapsed_slice_dims=(0,), start_index_map=(0,))
    return lax.gather(v, idx, dnums, (1,),
                      mode=lax.GatherScatterMode.PROMISE_IN_BOUNDS)


def _tile8(b):
    return jnp.tile(b.reshape(1, -1), (8, 1))


def _dotT(a, b):
    return lax.dot_general(a, b, (((1,), (1,)), ((), ())),
                           preferred_element_type=jnp.float32)


def _tc_matmul(xin, w, b8, relu, l2n):
    m, k = xin.shape
    dout = w.shape[0]
    bm = 2000
    grid = m // bm

    def body(x_ref, w_ref, b_ref, o_ref):
        t = _dotT(x_ref[...], w_ref[...]) + b_ref[0:1, :]
        if relu:
            t = jnp.maximum(t, 0.0)
        if l2n:
            nrm = jnp.sqrt(jnp.sum(t * t, axis=1, keepdims=True))
            t = t / jnp.maximum(nrm, 1e-12)
        o_ref[...] = t

    return pl.pallas_call(
        body,
        grid=(grid,),
        in_specs=[
            pl.BlockSpec((bm, k), lambda i: (i, 0)),
            pl.BlockSpec((dout, k), lambda i: (0, 0)),
            pl.BlockSpec((8, dout), lambda i: (0, 0)),
        ],
        out_specs=pl.BlockSpec((bm, dout), lambda i: (i, 0)),
        out_shape=jax.ShapeDtypeStruct((m, dout), jnp.float32),
    )(xin, w, b8)


def _tc_update(h, agg, ms, mn, extra):
    m, dh = h.shape
    bm = 2000
    grid = m // bm

    def body(h_ref, a_ref, ms_ref, mn_ref, e_ref, o_ref):
        t = _dotT(h_ref[...], ms_ref[...]) + _dotT(a_ref[...], mn_ref[...])
        t = t + e_ref[...]
        t = jnp.maximum(t, 0.0)
        nrm = jnp.sqrt(jnp.sum(t * t, axis=1, keepdims=True))
        o_ref[...] = t / jnp.maximum(nrm, 1e-12)

    return pl.pallas_call(
        body,
        grid=(grid,),
        in_specs=[
            pl.BlockSpec((bm, dh), lambda i: (i, 0)),
            pl.BlockSpec((bm, dh), lambda i: (i, 0)),
            pl.BlockSpec((dh, dh), lambda i: (0, 0)),
            pl.BlockSpec((dh, dh), lambda i: (0, 0)),
            pl.BlockSpec((bm, dh), lambda i: (i, 0)),
        ],
        out_specs=pl.BlockSpec((bm, dh), lambda i: (i, 0)),
        out_shape=jax.ShapeDtypeStruct((m, dh), jnp.float32),
    )(h, agg, ms, mn, extra)


def _tc_prep(ws, wn, wu, bs8, bn8, bu8):
    dh = ws.shape[0]

    def body(ws_ref, wn_ref, wu_ref, bs_ref, bn_ref, bu_ref,
             ms_ref, mn_ref, c_ref, d_ref):
        wua = wu_ref[:, :dh]
        wub = wu_ref[:, dh:]
        ms_ref[...] = lax.dot_general(wua, ws_ref[...],
                                      (((1,), (0,)), ((), ())),
                                      preferred_element_type=jnp.float32)
        mn_ref[...] = lax.dot_general(wub, wn_ref[...],
                                      (((1,), (0,)), ((), ())),
                                      preferred_element_type=jnp.float32)
        c_ref[...] = _dotT(bs_ref[...], wua) + bu_ref[...]
        d_ref[...] = _dotT(bn_ref[...], wub)

    return pl.pallas_call(
        body,
        out_shape=(
            jax.ShapeDtypeStruct((dh, dh), jnp.float32),
            jax.ShapeDtypeStruct((dh, dh), jnp.float32),
            jax.ShapeDtypeStruct((8, dh), jnp.float32),
            jax.ShapeDtypeStruct((8, dh), jnp.float32),
        ),
    )(ws, wn, wu, bs8, bn8, bu8)


def _sc_mesh():
    return plsc.VectorSubcoreMesh(core_axis_name="c", subcore_axis_name="s",
                                  num_cores=_NCORES, num_subcores=_NSUB)


def _make_bucket_kernel(e_total):
    t_outer = e_total // _SB
    g_inner = _SB // _LANES

    @functools.partial(
        pl.kernel,
        mesh=_sc_mesh(),
        compiler_params=pltpu.CompilerParams(needs_layout_passes=False),
        out_type=(
            jax.ShapeDtypeStruct((_NBUCKET * _CAP,), jnp.int32),
            jax.ShapeDtypeStruct((_NBUCKET * _CAP,), jnp.int32),
            jax.ShapeDtypeStruct((_NBUCKET * _LANES,), jnp.int32),
        ),
        scratch_types=[
            pltpu.VMEM((_SB,), jnp.int32),
            pltpu.VMEM((_SB,), jnp.int32),
            pltpu.VMEM((_CPW * _CAP + _LANES,), jnp.int32),
            pltpu.VMEM((_CPW * _CAP + _LANES,), jnp.int32),
            pltpu.VMEM((_CPW * _LANES,), jnp.int32),
        ],
    )
    def bucket(src_hbm, dst_hbm, bsrc_hbm, bdst_hbm, bcnt_hbm,
               sbuf, dbuf, bbs, bbd, cbuf):
        wid = lax.axis_index("s") * _NCORES + lax.axis_index("c")
        kc_base = wid * _CPW
        lane = lax.broadcasted_iota(jnp.int32, (_LANES,), 0)
        trash = _CPW * _CAP + lane

        def group(g, cnts):
            o = g * _LANES
            dstv = dbuf[pl.ds(o, _LANES)]
            srcv = sbuf[pl.ds(o, _LANES)]
            chunkv = jnp.right_shift(dstv, 6)
            dstloc = jnp.bitwise_and(dstv, _CHUNK - 1)
            out = []
            for k in range(_CPW):
                ck = cnts[k]
                m = chunkv == (kc_base + k)
                pref = plsc.cumsum(m.astype(jnp.int32))
                off = k * _CAP + jnp.minimum(ck, _CAP - _LANES)
                pos = jnp.where(m, off + pref - 1, trash)
                plsc.store_scatter(bbs, [pos], srcv)
                plsc.store_scatter(bbd, [pos], dstloc)
                pc = plsc.all_reduce_population_count(m)
                out.append(ck + pc)
            return tuple(out)

        def outer(t, cnts):
            pltpu.sync_copy(src_hbm.at[pl.ds(t * _SB, _SB)], sbuf)
            pltpu.sync_copy(dst_hbm.at[pl.ds(t * _SB, _SB)], dbuf)
            return lax.fori_loop(0, g_inner, group, cnts)

        zero16 = jnp.zeros((_LANES,), jnp.int32)
        cnts = lax.fori_loop(0, t_outer, outer, (zero16,) * _CPW)

        zdst = jnp.full((_LANES,), _DUMMY_DST, jnp.int32)
        for k in range(_CPW):
            ck = jnp.minimum(cnts[k], _CAP - _LANES)
            for t in range(8):
                pos = k * _CAP + jnp.minimum(ck + _LANES * t,
                                             _CAP - _LANES) + lane
                plsc.store_scatter(bbs, [pos], zero16)
                plsc.store_scatter(bbd, [pos], zdst)
            cbuf[pl.ds(k * _LANES, _LANES)] = ck

        pltpu.sync_copy(bbs.at[pl.ds(0, _CPW * _CAP)],
                        bsrc_hbm.at[pl.ds(wid * _CPW * _CAP, _CPW * _CAP)])
        pltpu.sync_copy(bbd.at[pl.ds(0, _CPW * _CAP)],
                        bdst_hbm.at[pl.ds(wid * _CPW * _CAP, _CPW * _CAP)])
        pltpu.sync_copy(cbuf, bcnt_hbm.at[pl.ds(wid * _CPW * _LANES,
                                                _CPW * _LANES)])

    return bucket


_DUMMY_DST = 8192
_DEG_LEN = _CPW * _CHUNK + _LANES


def _make_agg_kernel(n_nodes, dh, with_deg):
    gb = 32
    acc_len = (_CHUNK + 1) * dh

    out_type = [jax.ShapeDtypeStruct((_NPAD * dh,), jnp.float32)]
    if with_deg:
        out_type.append(jax.ShapeDtypeStruct((_NPAD,), jnp.float32))

    @functools.partial(
        pl.kernel,
        mesh=_sc_mesh(),
        compiler_params=pltpu.CompilerParams(needs_layout_passes=False),
        out_type=tuple(out_type),
        scratch_types=[
            pltpu.VMEM((_CAP,), jnp.int32),
            pltpu.VMEM((_CAP,), jnp.int32),
            pltpu.VMEM((gb, dh), jnp.float32),
            pltpu.VMEM((gb, dh), jnp.float32),
            pltpu.VMEM((acc_len,), jnp.float32),
            pltpu.VMEM((acc_len,), jnp.float32),
            pltpu.VMEM((_LANES,), jnp.int32),
            pltpu.VMEM((_DEG_LEN,), jnp.float32),
            pltpu.SemaphoreType.DMA,
            pltpu.SemaphoreType.DMA,
        ],
    )
    def agg(h_hbm, bsrc_hbm, bdst_hbm, bcnt_hbm, zeros_hbm,
            *out_and_scratch):
        if with_deg:
            agg_hbm, deg_hbm = out_and_scratch[0], out_and_scratch[1]
            rest = out_and_scratch[2:]
        else:
            agg_hbm = out_and_scratch[0]
            rest = out_and_scratch[1:]
        (ibuf, dbuf, rows0, rows1, acc_a, acc_b, cntbuf, degacc,
         sem0, sem1) = rest

        cid = lax.axis_index("c")
        sid = lax.axis_index("s")
        wid = sid * _NCORES + cid
        lane = lax.broadcasted_iota(jnp.int32, (_LANES,), 0)
        zeros16 = jnp.zeros((_LANES,), jnp.float32)
        ones16 = jnp.ones((_LANES,), jnp.float32)

        def start(b, rows, sem):
            off = jnp.minimum(b * gb, _CAP - gb)
            return pltpu.async_copy(h_hbm.at[ibuf.at[pl.ds(off, gb)]],
                                    rows, sem)

        def wait(rows, sem):
            pltpu.make_async_copy(h_hbm.at[ibuf.at[pl.ds(0, gb)]],
                                  rows, sem).wait()

        cconst = [lane + _LANES * c for c in range(dh // _LANES)]

        def accumulate(bbase, rows, k):
            for i in range(gb // _LANES):
                dv = dbuf[pl.ds(bbase + i * _LANES, _LANES)]
                dvm = jnp.minimum(dv, _CHUNK) * dh
                if with_deg:
                    di = jnp.minimum(k * _CHUNK + dv, _CPW * _CHUNK)
                    plsc.addupdate_scatter(degacc, [di], ones16)
                for j in range(_LANES):
                    off = pl.multiple_of(
                        jnp.sum(jnp.where(lane == j, dvm, 0)), dh)
                    tgt = acc_a if j % 2 == 0 else acc_b
                    r = i * _LANES + j
                    vals = [rows[r, pl.ds(c * _LANES, _LANES)]
                            for c in range(dh // _LANES)]
                    for c in range(dh // _LANES):
                        plsc.addupdate_scatter(
                            tgt.at[pl.ds(off + c * _LANES, _LANES)],
                            [lane], vals[c])

        if with_deg:
            for t in range(_DEG_LEN // _LANES):
                degacc[pl.ds(t * _LANES, _LANES)] = zeros16

        def do_bucket(k, _):
            bucket = wid * _CPW + k
            pltpu.sync_copy(bcnt_hbm.at[pl.ds(bucket * _LANES, _LANES)],
                            cntbuf)
            cnt = jnp.max(cntbuf[...])
            pltpu.sync_copy(bsrc_hbm.at[pl.ds(bucket * _CAP, _CAP)], ibuf)
            pltpu.sync_copy(bdst_hbm.at[pl.ds(bucket * _CAP, _CAP)], dbuf)
            pltpu.sync_copy(zeros_hbm, acc_a)
            pltpu.sync_copy(zeros_hbm, acc_b)
            npairs = (cnt + 2 * gb - 1) // (2 * gb)
            start(0, rows0, sem0)

            def pair(tt, _):
                wait(rows0, sem0)
                start(2 * tt + 1, rows1, sem1)
                accumulate(2 * tt * gb, rows0, k)
                wait(rows1, sem1)
                start(2 * tt + 2, rows0, sem0)
                accumulate((2 * tt + 1) * gb, rows1, k)
                return 0

            lax.fori_loop(0, npairs, pair, 0)
            wait(rows0, sem0)

            def merge(t, _):
                for c in range(16):
                    off = t * 256 + c * _LANES
                    acc_a[pl.ds(off, _LANES)] = (
                        acc_a[pl.ds(off, _LANES)] + acc_b[pl.ds(off, _LANES)])
                return 0

            lax.fori_loop(0, _CHUNK * dh // 256, merge, 0)
            pltpu.sync_copy(acc_a.at[pl.ds(0, _CHUNK * dh)],
                            agg_hbm.at[pl.ds(bucket * _CHUNK * dh,
                                             _CHUNK * dh)])
            return 0

        lax.fori_loop(0, _CPW, do_bucket, 0)
        if with_deg:
            pltpu.sync_copy(degacc.at[pl.ds(0, _CPW * _CHUNK)],
                            deg_hbm.at[pl.ds(wid * _CPW * _CHUNK,
                                             _CPW * _CHUNK)])

    return agg


def kernel(x, edge_index, W_in, b_in, Ws0, bs0, Wn0, bn0, Wu0, bu0,
           Ws1, bs1, Wn1, bn1, Wu1, bu1, W_out, b_out):
    n, din = x.shape
    dh = W_in.shape[0]
    e_total = edge_index.shape[1]

    src = edge_index[0]
    dst = edge_index[1]

    ms0, mn0, c0, d0 = _tc_prep(Ws0, Wn0, Wu0, _tile8(bs0), _tile8(bn0),
                                _tile8(bu0))
    ms1, mn1, c1, d1 = _tc_prep(Ws1, Wn1, Wu1, _tile8(bs1), _tile8(bn1),
                                _tile8(bu1))

    h = _tc_matmul(x, W_in, _tile8(b_in), relu=True, l2n=False)

    bsrc, bdst, bcnt = _make_bucket_kernel(e_total)(src, dst)

    zeros_acc = jnp.zeros(((_CHUNK + 1) * dh,), jnp.float32)

    agg0_flat, deg_pad = _make_agg_kernel(n, dh, True)(
        h, bsrc, bdst, bcnt, zeros_acc)
    agg0 = agg0_flat.reshape(_NPAD, dh)[:n]
    deg = deg_pad[:n]

    extra0 = deg[:, None] * d0[0] + c0[0]
    h = _tc_update(h, agg0, ms0, mn0, extra0)

    (agg1_flat,) = _make_agg_kernel(n, dh, False)(
        h, bsrc, bdst, bcnt, zeros_acc)
    agg1 = agg1_flat.reshape(_NPAD, dh)[:n]

    extra1 = deg[:, None] * d1[0] + c1[0]
    h = _tc_update(h, agg1, ms1, mn1, extra1)

    emb = _tc_matmul(h, W_out, _tile8(b_out), relu=False, l2n=True)
    return emb
